# baseline (speedup 1.0000x reference)
import functools

import jax
import jax.numpy as jnp
from jax import lax
from jax.experimental import pallas as pl
from jax.experimental.pallas import tpu as pltpu
from jax.experimental.pallas import tpu_sc as plsc

F_IN = 1000000
F_OUT = 16

NC = 2
NS = 16
NW = NC * NS

B = 16384 * 26
B_PER_W = B // NW
N_CHUNKS = 8
CHUNK = B_PER_W // N_CHUNKS


def _emb_body(x_hbm, table_hbm, out_hbm, idx_v, rows0, rows1, sem0, sem1):
    wid = lax.axis_index("s") * NC + lax.axis_index("c")
    base = wid * B_PER_W
    pltpu.sync_copy(x_hbm.at[pl.ds(base, B_PER_W)], idx_v)

    bufs = (rows0, rows1)
    sems = (sem0, sem1)

    descs = [None, None]
    for i in range(2):
        descs[i] = pltpu.async_copy(
            table_hbm.at[idx_v.at[pl.ds(i * CHUNK, CHUNK)]], bufs[i], sems[i])

    for i in range(N_CHUNKS):
        b = i % 2
        descs[b].wait()
        pltpu.sync_copy(bufs[b], out_hbm.at[pl.ds(base + i * CHUNK, CHUNK)])
        nxt = i + 2
        if nxt < N_CHUNKS:
            descs[b] = pltpu.async_copy(
                table_hbm.at[idx_v.at[pl.ds(nxt * CHUNK, CHUNK)]],
                bufs[b], sems[b])


@functools.partial(
    pl.kernel,
    out_type=jax.ShapeDtypeStruct((B, F_OUT), jnp.float32),
    mesh=plsc.VectorSubcoreMesh(core_axis_name="c", subcore_axis_name="s"),
    scratch_types=[
        pltpu.VMEM((B_PER_W,), jnp.int32),
        pltpu.VMEM((CHUNK, F_OUT), jnp.float32),
        pltpu.VMEM((CHUNK, F_OUT), jnp.float32),
        pltpu.SemaphoreType.DMA,
        pltpu.SemaphoreType.DMA,
    ],
    compiler_params=pltpu.CompilerParams(use_tc_tiling_on_sc=False),
)
def _emb_kernel(x_hbm, table_hbm, out_hbm, idx_v, rows0, rows1, sem0, sem1):
    _emb_body(x_hbm, table_hbm, out_hbm, idx_v, rows0, rows1, sem0, sem1)


def kernel(X, emb_weight):
    rows = _emb_kernel(X.reshape(-1), emb_weight)
    return rows.reshape(X.shape[0], X.shape[1], F_OUT)


# speedup vs baseline: 1.3015x; 1.3015x over previous
import functools

import jax
import jax.numpy as jnp
from jax import lax
from jax.experimental import pallas as pl
from jax.experimental.pallas import tpu as pltpu
from jax.experimental.pallas import tpu_sc as plsc

F_IN = 1000000
F_OUT = 16
NF = 26
NB = 16384

NC = 2
NS = 16
NW = NC * NS

ROWS_PER_W = NB // NW
N_CHUNKS = 8
CROWS = ROWS_PER_W // N_CHUNKS


def _emb_body(x_hbm, table_hbm, out_hbm, idx_v, rows0, rows1,
              gsem0, gsem1, wsem0, wsem1):
    wid = lax.axis_index("s") * NC + lax.axis_index("c")
    row_base = wid * ROWS_PER_W
    pltpu.sync_copy(x_hbm.at[pl.ds(row_base, ROWS_PER_W)], idx_v)

    bufs = (rows0, rows1)
    gsems = (gsem0, gsem1)
    wsems = (wsem0, wsem1)

    def fire_chunk(i, b):
        def body(j, _):
            pltpu.async_copy(
                table_hbm.at[idx_v.at[i * CROWS + j]], bufs[b].at[j], gsems[b])
            return 0
        lax.fori_loop(0, CROWS, body, 0)

    def drain_chunk(i, b):
        def body(j, _):
            pltpu.make_async_copy(
                table_hbm.at[idx_v.at[i * CROWS + j]], bufs[b].at[j],
                gsems[b]).wait()
            return 0
        lax.fori_loop(0, CROWS, body, 0)

    wdescs = [None, None]
    fire_chunk(0, 0)
    fire_chunk(1, 1)
    for i in range(N_CHUNKS):
        b = i % 2
        drain_chunk(i, b)
        wdescs[b] = pltpu.async_copy(
            bufs[b], out_hbm.at[pl.ds(row_base + i * CROWS, CROWS)], wsems[b])
        nxt = i + 2
        if nxt < N_CHUNKS:
            wdescs[b].wait()
            fire_chunk(nxt, b)
    wdescs[0].wait()
    wdescs[1].wait()


@functools.partial(
    pl.kernel,
    out_type=jax.ShapeDtypeStruct((NB, NF, F_OUT), jnp.float32),
    mesh=plsc.VectorSubcoreMesh(core_axis_name="c", subcore_axis_name="s"),
    scratch_types=[
        pltpu.VMEM((ROWS_PER_W, NF), jnp.int32),
        pltpu.VMEM((CROWS, NF, F_OUT), jnp.float32),
        pltpu.VMEM((CROWS, NF, F_OUT), jnp.float32),
        pltpu.SemaphoreType.DMA,
        pltpu.SemaphoreType.DMA,
        pltpu.SemaphoreType.DMA,
        pltpu.SemaphoreType.DMA,
    ],
    compiler_params=pltpu.CompilerParams(use_tc_tiling_on_sc=False),
)
def _emb_kernel(x_hbm, table_hbm, out_hbm, idx_v, rows0, rows1,
                gsem0, gsem1, wsem0, wsem1):
    _emb_body(x_hbm, table_hbm, out_hbm, idx_v, rows0, rows1,
              gsem0, gsem1, wsem0, wsem1)


def kernel(X, emb_weight):
    return _emb_kernel(X, emb_weight)


# speedup vs baseline: 1.6928x; 1.3007x over previous
import functools

import jax
import jax.numpy as jnp
from jax import lax
from jax.experimental import pallas as pl
from jax.experimental.pallas import tpu as pltpu
from jax.experimental.pallas import tpu_sc as plsc

F_IN = 1000000
F_OUT = 16
NF = 26
NB = 16384

NC = 2
NS = 16
NW = NC * NS

BPW = NB // NW
TCW = BPW // 128


def _emb_body(xt_hbm, table_hbm, out_hbm, idx_v, buf0, buf1, st0, st1,
              gsem0, gsem1, wsem0, wsem1):
    wid = lax.axis_index("s") * NC + lax.axis_index("c")
    base = wid * BPW
    pltpu.sync_copy(xt_hbm.at[:, pl.ds(base, BPW)], idx_v)

    bufs = (buf0, buf1)
    stages = (st0, st1)
    gsems = (gsem0, gsem1)
    wsems = (wsem0, wsem1)

    iota = lax.iota(jnp.int32, 16)
    d_idx = [jnp.full((16,), d, jnp.int32) for d in range(F_OUT)]

    def fire_gather(f):
        return pltpu.async_copy(
            table_hbm.at[idx_v.at[f]], bufs[f % 2], gsems[f % 2])

    def transpose_block(p):
        buf, stage = bufs[p], stages[p]

        def body(l16, _):
            b_idx = iota + l16 * 16
            tcl = l16 // 8
            lo = (l16 % 8) * 16
            for d in range(F_OUT):
                v = plsc.load_gather(buf, [b_idx, d_idx[d]])
                stage[d // 8, tcl, d % 8, pl.ds(lo, 16)] = v
            return 0

        lax.fori_loop(0, BPW // 16, body, 0)

    def fire_writeout(f, p):
        return pltpu.async_copy(
            stages[p],
            out_hbm.at[f, :, pl.ds(wid * TCW, TCW)],
            wsems[p])

    gdescs = [None, None]
    wdescs = [None, None]
    gdescs[0] = fire_gather(0)
    gdescs[1] = fire_gather(1)
    for f in range(NF):
        p = f % 2
        gdescs[p].wait()
        if wdescs[p] is not None:
            wdescs[p].wait()
        transpose_block(p)
        if f + 2 < NF:
            gdescs[p] = fire_gather(f + 2)
        wdescs[p] = fire_writeout(f, p)
    wdescs[0].wait()
    wdescs[1].wait()


@functools.partial(
    pl.kernel,
    out_type=jax.ShapeDtypeStruct((NF, 2, 128, 8, 128), jnp.float32),
    mesh=plsc.VectorSubcoreMesh(core_axis_name="c", subcore_axis_name="s"),
    scratch_types=[
        pltpu.VMEM((NF, BPW), jnp.int32),
        pltpu.VMEM((BPW, F_OUT), jnp.float32),
        pltpu.VMEM((BPW, F_OUT), jnp.float32),
        pltpu.VMEM((2, TCW, 8, 128), jnp.float32),
        pltpu.VMEM((2, TCW, 8, 128), jnp.float32),
        pltpu.SemaphoreType.DMA,
        pltpu.SemaphoreType.DMA,
        pltpu.SemaphoreType.DMA,
        pltpu.SemaphoreType.DMA,
    ],
    compiler_params=pltpu.CompilerParams(
        use_tc_tiling_on_sc=False, needs_layout_passes=False),
)
def _emb_kernel(xt_hbm, table_hbm, out_hbm, idx_v, buf0, buf1, st0, st1,
                gsem0, gsem1, wsem0, wsem1):
    _emb_body(xt_hbm, table_hbm, out_hbm, idx_v, buf0, buf1, st0, st1,
              gsem0, gsem1, wsem0, wsem1)


def kernel(X, emb_weight):
    out5 = _emb_kernel(X.T, emb_weight)
    return out5.transpose(2, 4, 0, 1, 3).reshape(NB, NF, F_OUT)
